# baseline (speedup 1.0000x reference)
import functools

import jax
import jax.numpy as jnp
from jax import lax
from jax.experimental import pallas as pl
from jax.experimental.pallas import tpu as pltpu
from jax.experimental.pallas import tpu_sc as plsc

D = 64
GRP = 128
CHUNK = 4
SUPER = 32
NC, NS = 2, 16
NW = NC * NS
ROWS = CHUNK * GRP


def _body(idx_hbm, table_hbm, out_hbm, idx_v, rows_v, gsem, osem, *, groups):
    groups_per_w = groups // NW
    wid = lax.axis_index("s") * NC + lax.axis_index("c")
    g0 = wid * groups_per_w
    cps = SUPER // CHUNK
    n_chunks = groups_per_w // CHUNK

    def fire_gathers(c, ib, j):
        for k in range(CHUNK):
            pltpu.async_copy(
                table_hbm.at[idx_v.at[ib, j * CHUNK + k]],
                rows_v.at[c % 2, pl.ds(k * GRP, GRP)],
                gsem,
            )

    def wait_gathers(c):
        pltpu.make_async_copy(
            out_hbm.at[pl.ds(0, ROWS)], rows_v.at[c % 2], gsem
        ).wait()

    def fire_write(c):
        pltpu.async_copy(
            rows_v.at[c % 2],
            out_hbm.at[pl.ds((g0 + c * CHUNK) * GRP, ROWS)],
            osem,
        )

    def drain_write(c):
        pltpu.make_async_copy(
            rows_v.at[c % 2], out_hbm.at[pl.ds(0, ROWS)], osem
        ).wait()

    def outer(s, _):
        ib = s % 2
        pltpu.sync_copy(idx_hbm.at[pl.ds(g0 + s * SUPER, SUPER)], idx_v.at[ib])

        def inner(j, _):
            c = s * cps + j

            @pl.when(c >= 2)
            def _():
                drain_write(c)

            fire_gathers(c, ib, j)

            @pl.when(c >= 1)
            def _():
                wait_gathers(c - 1)
                fire_write(c - 1)

            return 0

        lax.fori_loop(0, cps, inner, 0)
        return 0

    lax.fori_loop(0, groups_per_w // SUPER, outer, 0)
    wait_gathers(n_chunks - 1)
    fire_write(n_chunks - 1)
    drain_write(n_chunks - 2)
    drain_write(n_chunks - 1)


@functools.partial(jax.jit, static_argnames=("groups",))
def _gather(idx, table, *, groups):
    body = functools.partial(_body, groups=groups)
    return pl.kernel(
        body,
        out_type=jax.ShapeDtypeStruct((groups * GRP, D), jnp.float32),
        mesh=plsc.VectorSubcoreMesh(core_axis_name="c", subcore_axis_name="s"),
        scratch_types=[
            pltpu.VMEM((2, SUPER, GRP), jnp.int32),
            pltpu.VMEM((2, ROWS, D), jnp.float32),
            pltpu.SemaphoreType.DMA,
            pltpu.SemaphoreType.DMA,
        ],
        compiler_params=pltpu.CompilerParams(use_tc_tiling_on_sc=False),
    )(idx, table)


def kernel(x, W):
    b, h = x.shape
    groups = (b * h) // GRP
    idx = x.reshape(groups, GRP).astype(jnp.int32)
    out = _gather(idx, W, groups=groups)
    return out.reshape(b, h, D)


# speedup vs baseline: 1.0403x; 1.0403x over previous
import functools

import jax
import jax.numpy as jnp
from jax import lax
from jax.experimental import pallas as pl
from jax.experimental.pallas import tpu as pltpu
from jax.experimental.pallas import tpu_sc as plsc

D = 64
GRP = 100
CHUNK = 4
SUPER = 32
NC, NS = 2, 16
NW = NC * NS
BPC = CHUNK * GRP // 200


def _body(idx_hbm, table_hbm, out_hbm, idx_v, rows_v, gsem, osem, *, groups):
    groups_per_w = groups // NW
    wid = lax.axis_index("s") * NC + lax.axis_index("c")
    g0 = wid * groups_per_w
    b0w = wid * (groups_per_w * GRP // 200)
    cps = SUPER // CHUNK
    n_chunks = groups_per_w // CHUNK

    def fire_gathers(c, ib, j):
        for k in range(CHUNK):
            pltpu.async_copy(
                table_hbm.at[idx_v.at[ib, j * CHUNK + k]],
                rows_v.at[c % 2, k // 2, pl.ds((k % 2) * GRP, GRP)],
                gsem,
            )

    def wait_gathers(c):
        pltpu.make_async_copy(
            out_hbm.at[pl.ds(0, BPC)], rows_v.at[c % 2], gsem
        ).wait()

    def fire_write(c):
        pltpu.async_copy(
            rows_v.at[c % 2],
            out_hbm.at[pl.ds(b0w + c * BPC, BPC)],
            osem,
        )

    def drain_write(c):
        pltpu.make_async_copy(
            rows_v.at[c % 2], out_hbm.at[pl.ds(0, BPC)], osem
        ).wait()

    def outer(s, _):
        ib = s % 2
        pltpu.sync_copy(idx_hbm.at[pl.ds(g0 + s * SUPER, SUPER)], idx_v.at[ib])

        def inner(j, _):
            c = s * cps + j

            @pl.when(c >= 2)
            def _():
                drain_write(c)

            fire_gathers(c, ib, j)

            @pl.when(c >= 1)
            def _():
                wait_gathers(c - 1)
                fire_write(c - 1)

            return 0

        lax.fori_loop(0, cps, inner, 0)
        return 0

    lax.fori_loop(0, groups_per_w // SUPER, outer, 0)
    wait_gathers(n_chunks - 1)
    fire_write(n_chunks - 1)
    drain_write(n_chunks - 2)
    drain_write(n_chunks - 1)


@functools.partial(jax.jit, static_argnames=("groups", "batch", "hist"))
def _gather(idx, table, *, groups, batch, hist):
    body = functools.partial(_body, groups=groups)
    return pl.kernel(
        body,
        out_type=jax.ShapeDtypeStruct((batch, hist, D), jnp.float32),
        mesh=plsc.VectorSubcoreMesh(core_axis_name="c", subcore_axis_name="s"),
        scratch_types=[
            pltpu.VMEM((2, SUPER, GRP), jnp.int32),
            pltpu.VMEM((2, BPC, 200, D), jnp.float32),
            pltpu.SemaphoreType.DMA,
            pltpu.SemaphoreType.DMA,
        ],
        compiler_params=pltpu.CompilerParams(use_tc_tiling_on_sc=False),
    )(idx, table)


def kernel(x, W):
    b, h = x.shape
    groups = (b * h) // GRP
    idx = x.reshape(groups, GRP).astype(jnp.int32)
    return _gather(idx, W, groups=groups, batch=b, hist=h)
